# baseline (speedup 1.0000x reference)
import functools

import jax
import jax.numpy as jnp
from jax import lax
from jax.experimental import pallas as pl
from jax.experimental.pallas import tpu as pltpu
from jax.experimental.pallas import tpu_sc as plsc

NC = 2
NS = 16
NW = NC * NS


def _mlp_body(x_ref, w1_ref, b1_ref, w2_ref, b2_ref, o_ref):
    x = x_ref[...]
    h = lax.dot_general(x, w1_ref[...], (((1,), (1,)), ((), ())),
                        preferred_element_type=jnp.float32)
    h = h + b1_ref[...]
    h = jnp.where(h >= 0, h, 0.01 * h)
    o = lax.dot_general(h, w2_ref[...], (((1,), (1,)), ((), ())),
                        preferred_element_type=jnp.float32)
    o_ref[...] = o + b2_ref[...]


def kernel(memory, nodes, W1, b1, W2, b2):
    M, D = memory.shape
    B = nodes.shape[0]
    Hf = W1.shape[0]

    bpw = B // NW
    stripe = (M // NW) // 8 * 8
    tail = M - stripe * NW
    nodes2 = nodes.reshape(NW, bpw)

    mesh = plsc.VectorSubcoreMesh(core_axis_name="c", subcore_axis_name="s")
    sc_params = pltpu.CompilerParams(needs_layout_passes=False)

    CR = 200
    NBUF = 4
    n_cc = M // CR
    npw = -(-n_cc // NW)
    npw_pad = -(-npw // NBUF) * NBUF

    @functools.partial(
        pl.kernel, mesh=mesh, compiler_params=sc_params,
        out_type=jax.ShapeDtypeStruct((M, D), jnp.float32),
        scratch_types=(
            [pltpu.VMEM((CR, D), jnp.float32)] * NBUF
            + [pltpu.SemaphoreType.DMA] * NBUF
            + [pltpu.SemaphoreType.DMA] * NBUF
        ),
    )
    def clone_k(mem_hbm, out_hbm, *scr):
        bufs = scr[:NBUF]
        sin = scr[NBUF:2 * NBUF]
        sout = scr[2 * NBUF:]
        wid = lax.axis_index("s") * NC + lax.axis_index("c")

        for b in range(NBUF):
            c = b * NW + wid

            @pl.when(c < n_cc)
            def _(c=c, b=b):
                pltpu.make_async_copy(
                    mem_hbm.at[pl.ds(c * CR, CR)], bufs[b], sin[b]
                ).start()

        @pl.loop(0, npw_pad, step=NBUF)
        def _(k0):
            for b in range(NBUF):
                k = k0 + b
                c = k * NW + wid

                @pl.when(c < n_cc)
                def _(k=k, b=b, c=c):
                    pltpu.make_async_copy(
                        mem_hbm.at[pl.ds(0, CR)], bufs[b], sin[b]
                    ).wait()
                    pltpu.make_async_copy(
                        bufs[b], out_hbm.at[pl.ds(c * CR, CR)], sout[b]
                    ).start()

                cn = (k + NBUF) * NW + wid

                @pl.when(cn < n_cc)
                def _(k=k, b=b, cn=cn):
                    pltpu.make_async_copy(
                        bufs[b], out_hbm.at[pl.ds(0, CR)], sout[b]
                    ).wait()
                    pltpu.make_async_copy(
                        mem_hbm.at[pl.ds(cn * CR, CR)], bufs[b], sin[b]
                    ).start()

        for b in range(NBUF):
            c = b * NW + wid

            @pl.when(c < n_cc)
            def _(b=b):
                pltpu.make_async_copy(
                    bufs[b], out_hbm.at[pl.ds(0, CR)], sout[b]
                ).wait()

    @functools.partial(
        pl.kernel, mesh=mesh, compiler_params=sc_params,
        out_type=jax.ShapeDtypeStruct((B, D), jnp.float32),
        scratch_types=[
            pltpu.VMEM((bpw,), jnp.int32),
            pltpu.VMEM((bpw, D), jnp.float32),
            pltpu.SemaphoreType.DMA,
        ],
    )
    def gather_k(mem_hbm, idx_hbm, sel_hbm, idx_v, rows_v, sem):
        wid = lax.axis_index("s") * NC + lax.axis_index("c")
        pltpu.sync_copy(idx_hbm.at[wid], idx_v)
        lanes = lax.iota(jnp.int32, 16)

        @pl.loop(0, bpw // 16)
        def _(c):
            chunk = idx_v[pl.ds(c * 16, 16)]
            for lane in range(16):
                k = jnp.max(jnp.where(lanes == lane, chunk, 0))
                pltpu.make_async_copy(
                    mem_hbm.at[pl.ds(k, 1)],
                    rows_v.at[pl.ds(c * 16 + lane, 1)],
                    sem,
                ).start()

        @pl.loop(0, bpw)
        def _(i):
            pltpu.make_async_copy(
                mem_hbm.at[pl.ds(0, 1)], rows_v.at[pl.ds(0, 1)], sem
            ).wait()

        pltpu.sync_copy(rows_v, sel_hbm.at[pl.ds(wid * bpw, bpw)])

    @functools.partial(
        pl.kernel, mesh=mesh, compiler_params=sc_params,
        out_type=(),
        scratch_types=[
            pltpu.VMEM((bpw,), jnp.int32),
            pltpu.VMEM((bpw, D), jnp.float32),
            pltpu.SemaphoreType.DMA,
            pltpu.SemaphoreType.DMA,
        ],
    )
    def scatter_k(idx_hbm, rows_hbm, out_hbm, idx_v, rows_v, sem, sem2):
        wid = lax.axis_index("s") * NC + lax.axis_index("c")
        pltpu.sync_copy(idx_hbm.at[wid], idx_v)
        pltpu.sync_copy(rows_hbm.at[pl.ds(wid * bpw, bpw)], rows_v)
        lanes = lax.iota(jnp.int32, 16)

        @pl.loop(0, bpw // 16)
        def _(c):
            chunk = idx_v[pl.ds(c * 16, 16)]
            for lane in range(16):
                k = jnp.max(jnp.where(lanes == lane, chunk, 0))
                pltpu.make_async_copy(
                    rows_v.at[pl.ds(c * 16 + lane, 1)],
                    out_hbm.at[pl.ds(k, 1)],
                    sem2,
                ).start()

        @pl.loop(0, bpw)
        def _(i):
            pltpu.make_async_copy(
                out_hbm.at[pl.ds(0, 1)], rows_v.at[pl.ds(0, 1)], sem2
            ).wait()

    cloned = clone_k(memory)
    sel = gather_k(memory, nodes2)
    mlp_rows = pl.pallas_call(
        _mlp_body,
        out_shape=jax.ShapeDtypeStruct((B, D), jnp.float32),
    )(sel, W1, b1.reshape(1, Hf), W2, b2.reshape(1, D))

    out_ref = jax.new_ref(cloned)
    scatter_k(nodes2, mlp_rows, out_ref)
    return out_ref[...]


# speedup vs baseline: 1.4944x; 1.4944x over previous
import functools

import jax
import jax.numpy as jnp
from jax import lax
from jax.experimental import pallas as pl
from jax.experimental.pallas import tpu as pltpu
from jax.experimental.pallas import tpu_sc as plsc

NC = 2
NS = 16
NW = NC * NS


def _mlp_body(x_ref, w1_ref, b1_ref, w2_ref, b2_ref, o_ref):
    x = x_ref[...]
    h = lax.dot_general(x, w1_ref[...], (((1,), (1,)), ((), ())),
                        preferred_element_type=jnp.float32)
    h = h + b1_ref[...]
    h = jnp.where(h >= 0, h, 0.01 * h)
    o = lax.dot_general(h, w2_ref[...], (((1,), (1,)), ((), ())),
                        preferred_element_type=jnp.float32)
    o_ref[...] = o + b2_ref[...]


def kernel(memory, nodes, W1, b1, W2, b2):
    M, D = memory.shape
    B = nodes.shape[0]
    Hf = W1.shape[0]

    bpw = B // NW
    stripe = (M // NW) // 8 * 8
    tail = M - stripe * NW
    nodes2 = nodes.reshape(NW, bpw)

    mesh = plsc.VectorSubcoreMesh(core_axis_name="c", subcore_axis_name="s")
    sc_params = pltpu.CompilerParams(needs_layout_passes=False)

    CR = 200
    NBUF = 4
    n_cc = M // CR
    npw = -(-n_cc // NW)
    npw_pad = -(-npw // NBUF) * NBUF

    @functools.partial(
        pl.kernel, mesh=mesh, compiler_params=sc_params,
        out_type=jax.ShapeDtypeStruct((M, D), jnp.float32),
        scratch_types=(
            [pltpu.VMEM((CR, D), jnp.float32)] * NBUF
            + [pltpu.SemaphoreType.DMA] * NBUF
            + [pltpu.SemaphoreType.DMA] * NBUF
        ),
    )
    def clone_k(mem_hbm, out_hbm, *scr):
        bufs = scr[:NBUF]
        sin = scr[NBUF:2 * NBUF]
        sout = scr[2 * NBUF:]
        wid = lax.axis_index("s") * NC + lax.axis_index("c")

        for b in range(NBUF):
            c = b * NW + wid

            @pl.when(c < n_cc)
            def _(c=c, b=b):
                pltpu.make_async_copy(
                    mem_hbm.at[pl.ds(c * CR, CR)], bufs[b], sin[b]
                ).start()

        @pl.loop(0, npw_pad, step=NBUF)
        def _(k0):
            for b in range(NBUF):
                k = k0 + b
                c = k * NW + wid

                @pl.when(c < n_cc)
                def _(k=k, b=b, c=c):
                    pltpu.make_async_copy(
                        mem_hbm.at[pl.ds(0, CR)], bufs[b], sin[b]
                    ).wait()
                    pltpu.make_async_copy(
                        bufs[b], out_hbm.at[pl.ds(c * CR, CR)], sout[b]
                    ).start()

                cn = (k + NBUF) * NW + wid

                @pl.when(cn < n_cc)
                def _(k=k, b=b, cn=cn):
                    pltpu.make_async_copy(
                        bufs[b], out_hbm.at[pl.ds(0, CR)], sout[b]
                    ).wait()
                    pltpu.make_async_copy(
                        mem_hbm.at[pl.ds(cn * CR, CR)], bufs[b], sin[b]
                    ).start()

        for b in range(NBUF):
            c = b * NW + wid

            @pl.when(c < n_cc)
            def _(b=b):
                pltpu.make_async_copy(
                    bufs[b], out_hbm.at[pl.ds(0, CR)], sout[b]
                ).wait()

    @functools.partial(
        pl.kernel, mesh=mesh, compiler_params=sc_params,
        out_type=jax.ShapeDtypeStruct((B, D), jnp.float32),
        scratch_types=[
            pltpu.VMEM((bpw,), jnp.int32),
            pltpu.VMEM((bpw, D), jnp.float32),
            pltpu.SemaphoreType.DMA,
        ],
    )
    def gather_k(mem_hbm, idx_hbm, sel_hbm, idx_v, rows_v, sem):
        wid = lax.axis_index("s") * NC + lax.axis_index("c")
        pltpu.sync_copy(idx_hbm.at[wid], idx_v)
        lanes = lax.iota(jnp.int32, 16)

        @pl.loop(0, bpw // 16)
        def _(c):
            chunk = idx_v[pl.ds(c * 16, 16)]
            for lane in range(16):
                k = jnp.max(jnp.where(lanes == lane, chunk, 0))
                pltpu.make_async_copy(
                    mem_hbm.at[pl.ds(k, 1)],
                    rows_v.at[pl.ds(c * 16 + lane, 1)],
                    sem,
                ).start()

        @pl.loop(0, bpw)
        def _(i):
            pltpu.make_async_copy(
                mem_hbm.at[pl.ds(0, 1)], rows_v.at[pl.ds(0, 1)], sem
            ).wait()

        pltpu.sync_copy(rows_v, sel_hbm.at[pl.ds(wid * bpw, bpw)])

    @functools.partial(
        pl.kernel, mesh=mesh, compiler_params=sc_params,
        out_type=(),
        scratch_types=[
            pltpu.VMEM((bpw,), jnp.int32),
            pltpu.VMEM((bpw, D), jnp.float32),
            pltpu.SemaphoreType.DMA,
            pltpu.SemaphoreType.DMA,
        ],
    )
    def scatter_k(idx_hbm, rows_hbm, out_hbm, idx_v, rows_v, sem, sem2):
        wid = lax.axis_index("s") * NC + lax.axis_index("c")
        pltpu.sync_copy(idx_hbm.at[wid], idx_v)
        pltpu.sync_copy(rows_hbm.at[pl.ds(wid * bpw, bpw)], rows_v)
        lanes = lax.iota(jnp.int32, 16)

        @pl.loop(0, bpw // 16)
        def _(c):
            chunk = idx_v[pl.ds(c * 16, 16)]
            for lane in range(16):
                k = jnp.max(jnp.where(lanes == lane, chunk, 0))
                pltpu.make_async_copy(
                    rows_v.at[pl.ds(c * 16 + lane, 1)],
                    out_hbm.at[pl.ds(k, 1)],
                    sem2,
                ).start()

        @pl.loop(0, bpw)
        def _(i):
            pltpu.make_async_copy(
                out_hbm.at[pl.ds(0, 1)], rows_v.at[pl.ds(0, 1)], sem2
            ).wait()

    sel = gather_k(memory, nodes2)
    mlp_rows = pl.pallas_call(
        _mlp_body,
        out_shape=jax.ShapeDtypeStruct((B, D), jnp.float32),
    )(sel, W1, b1.reshape(1, Hf), W2, b2.reshape(1, D))

    out_ref = jax.new_ref(memory)
    scatter_k(nodes2, mlp_rows, out_ref)
    return out_ref[...]
